# Initial kernel scaffold; baseline (speedup 1.0000x reference)
import jax
import jax.numpy as jnp
from jax.experimental import pallas as pl


def kernel(inputs, token_table, pos_table):
    raise NotImplementedError("write your pallas kernel here")


# speedup vs baseline: 5.0259x; 5.0259x over previous
import functools

import jax
import jax.numpy as jnp
from jax import lax
from jax.experimental import pallas as pl
from jax.experimental.pallas import tpu as pltpu
from jax.experimental.pallas import tpu_sc as plsc

BATCH = 4096
SEQ = 200
VOCAB = 100000
D = 64
LANES = 16

NC = 2
NS = 16
NW = NC * NS

TOK = BATCH * SEQ
TOK_W = TOK // NW
CHUNK = 2 * SEQ
NCHUNK = TOK_W // CHUNK
GSUB = 8
GLEN = CHUNK // GSUB
IDX2_ROWS = TOK // GLEN


def _sc_body(idx2_hbm, idxf_hbm, table_hbm, pos_hbm, out_hbm,
             idx2_v, idxf_v, pos_v, rows_v, sem):
    wid = lax.axis_index("s") * NC + lax.axis_index("c")
    tok0 = wid * TOK_W

    pltpu.sync_copy(pos_hbm, pos_v)

    def chunk_body(c, carry):
        base = pl.multiple_of(tok0 + c * CHUNK, CHUNK)

        pltpu.sync_copy(idx2_hbm.at[pl.ds(pl.multiple_of(base // GLEN, GSUB),
                                          GSUB)], idx2_v)
        pltpu.sync_copy(idxf_hbm.at[pl.ds(base, CHUNK)],
                        idxf_v.at[pl.ds(0, CHUNK)])

        cps = [
            pltpu.async_copy(table_hbm.at[idx2_v.at[j]],
                             rows_v.at[pl.ds(j * GLEN, GLEN)], sem)
            for j in range(GSUB)
        ]
        for cp in cps:
            cp.wait()

        zero = jnp.zeros((LANES,), jnp.float32)

        def pos_body(s, carry2):
            i0 = idxf_v[pl.ds(s, LANES)][0]
            i1 = idxf_v[pl.ds(s + SEQ, LANES)][0]

            @pl.when(i0 == 0)
            def _z0():
                for q in range(D // LANES):
                    rows_v[s, pl.ds(q * LANES, LANES)] = zero

            @pl.when(i1 == 0)
            def _z1():
                for q in range(D // LANES):
                    rows_v[s + SEQ, pl.ds(q * LANES, LANES)] = zero

            for q in range(D // LANES):
                sl = pl.ds(q * LANES, LANES)
                pv = pos_v[s, sl]
                plsc.addupdate(rows_v.at[s, sl], pv)
                plsc.addupdate(rows_v.at[s + SEQ, sl], pv)
            return carry2

        lax.fori_loop(0, SEQ, pos_body, 0)

        pltpu.sync_copy(rows_v, out_hbm.at[pl.ds(base, CHUNK)])
        return carry

    lax.fori_loop(0, NCHUNK, chunk_body, 0)


@jax.jit
def _sc_embed(idx2, idxf, token_table, pos_table):
    mesh = plsc.VectorSubcoreMesh(core_axis_name="c", subcore_axis_name="s")
    run = functools.partial(
        pl.kernel,
        out_type=jax.ShapeDtypeStruct((TOK, D), jnp.float32),
        mesh=mesh,
        compiler_params=pltpu.CompilerParams(use_tc_tiling_on_sc=False),
        scratch_types=[
            pltpu.VMEM((GSUB, GLEN), jnp.int32),
            pltpu.VMEM((CHUNK + LANES,), jnp.int32),
            pltpu.VMEM((SEQ, D), jnp.float32),
            pltpu.VMEM((CHUNK, D), jnp.float32),
            pltpu.SemaphoreType.DMA,
        ],
    )(_sc_body)
    return run(idx2, idxf, token_table, pos_table)


def kernel(inputs, token_table, pos_table):
    idxf = inputs.reshape(TOK).astype(jnp.int32)
    idx2 = idxf.reshape(IDX2_ROWS, GLEN)
    out = _sc_embed(idx2, idxf, token_table, pos_table)
    return out.reshape(BATCH, SEQ, D)


# speedup vs baseline: 6.3374x; 1.2609x over previous
import functools

import jax
import jax.numpy as jnp
from jax import lax
from jax.experimental import pallas as pl
from jax.experimental.pallas import tpu as pltpu
from jax.experimental.pallas import tpu_sc as plsc

BATCH = 4096
SEQ = 200
VOCAB = 100000
D = 64
LANES = 16

NC = 2
NS = 16
NW = NC * NS

TOK = BATCH * SEQ
TOK_W = TOK // NW
CHUNK = 2 * SEQ
NCHUNK = TOK_W // CHUNK
GSUB = 8
GLEN = CHUNK // GSUB
IDX2_ROWS = TOK // GLEN
NBUF = 4


def _sc_body(idx2_hbm, idxf_hbm, table_hbm, pos_hbm, out_hbm, *s):
    idx2_b = s[0:NBUF]
    idxf_b = s[NBUF:2 * NBUF]
    pos_v = s[2 * NBUF]
    rows_b = s[2 * NBUF + 1:3 * NBUF + 1]
    gsem = s[3 * NBUF + 1:4 * NBUF + 1]
    osem = s[4 * NBUF + 1:5 * NBUF + 1]
    isem = s[5 * NBUF + 1:6 * NBUF + 1]

    wid = lax.axis_index("s") * NC + lax.axis_index("c")
    tok0 = wid * TOK_W

    def chunk_base(c):
        return pl.multiple_of(tok0 + c * CHUNK, CHUNK)

    def fire_idx(c, b):
        base = chunk_base(c)
        pltpu.async_copy(
            idx2_hbm.at[pl.ds(pl.multiple_of(base // GLEN, GSUB), GSUB)],
            idx2_b[b], isem[b])
        pltpu.async_copy(idxf_hbm.at[pl.ds(base, CHUNK)],
                         idxf_b[b].at[pl.ds(0, CHUNK)], isem[b])

    def wait_idx(b):
        pltpu.make_async_copy(idx2_hbm.at[pl.ds(0, GSUB)],
                              idx2_b[b], isem[b]).wait()
        pltpu.make_async_copy(idxf_hbm.at[pl.ds(0, CHUNK)],
                              idxf_b[b].at[pl.ds(0, CHUNK)], isem[b]).wait()

    def fire_gather(b):
        for j in range(GSUB):
            pltpu.async_copy(table_hbm.at[idx2_b[b].at[j]],
                             rows_b[b].at[pl.ds(j * GLEN, GLEN)], gsem[b])

    def wait_gather(b):
        pltpu.make_async_copy(out_hbm.at[pl.ds(0, CHUNK)],
                              rows_b[b], gsem[b]).wait()

    def fire_out(c, b):
        pltpu.async_copy(rows_b[b], out_hbm.at[pl.ds(chunk_base(c), CHUNK)],
                         osem[b])

    def wait_out(b):
        pltpu.make_async_copy(rows_b[b], out_hbm.at[pl.ds(0, CHUNK)],
                              osem[b]).wait()

    def compute(b):
        idxf_v = idxf_b[b]
        rows_v = rows_b[b]
        zero = jnp.zeros((LANES,), jnp.float32)

        def pos_body(t, carry2):
            i0 = idxf_v[pl.ds(t, LANES)][0]
            i1 = idxf_v[pl.ds(t + SEQ, LANES)][0]

            @pl.when(i0 == 0)
            def _z0():
                for q in range(D // LANES):
                    rows_v[t, pl.ds(q * LANES, LANES)] = zero

            @pl.when(i1 == 0)
            def _z1():
                for q in range(D // LANES):
                    rows_v[t + SEQ, pl.ds(q * LANES, LANES)] = zero

            for q in range(D // LANES):
                sl = pl.ds(q * LANES, LANES)
                pv = pos_v[t, sl]
                plsc.addupdate(rows_v.at[t, sl], pv)
                plsc.addupdate(rows_v.at[t + SEQ, sl], pv)
            return carry2

        lax.fori_loop(0, SEQ, pos_body, 0)

    pltpu.sync_copy(pos_hbm, pos_v)

    fire_idx(0, 0)
    fire_idx(1, 1)
    wait_idx(0)
    fire_gather(0)

    def outer(k, carry):
        for b in range(NBUF):
            c = k * NBUF + b

            @pl.when(c + 2 < NCHUNK)
            def _s1():
                fire_idx(c + 2, (b + 2) % NBUF)

            @pl.when(c + 1 < NCHUNK)
            def _s2():
                @pl.when(c >= 3)
                def _drain():
                    wait_out((b + 1) % NBUF)

                wait_idx((b + 1) % NBUF)
                fire_gather((b + 1) % NBUF)

            wait_gather(b)
            compute(b)
            fire_out(c, b)
        return carry

    lax.fori_loop(0, NCHUNK // NBUF, outer, 0)

    for b in range(NBUF):
        wait_out(b)


@jax.jit
def _sc_embed(idx2, idxf, token_table, pos_table):
    mesh = plsc.VectorSubcoreMesh(core_axis_name="c", subcore_axis_name="s")
    run = functools.partial(
        pl.kernel,
        out_type=jax.ShapeDtypeStruct((TOK, D), jnp.float32),
        mesh=mesh,
        compiler_params=pltpu.CompilerParams(use_tc_tiling_on_sc=False),
        scratch_types=(
            [pltpu.VMEM((GSUB, GLEN), jnp.int32)] * NBUF
            + [pltpu.VMEM((CHUNK + LANES,), jnp.int32)] * NBUF
            + [pltpu.VMEM((SEQ, D), jnp.float32)]
            + [pltpu.VMEM((CHUNK, D), jnp.float32)] * NBUF
            + [pltpu.SemaphoreType.DMA] * (3 * NBUF)
        ),
    )(_sc_body)
    return run(idx2, idxf, token_table, pos_table)


def kernel(inputs, token_table, pos_table):
    idxf = inputs.reshape(TOK).astype(jnp.int32)
    idx2 = idxf.reshape(IDX2_ROWS, GLEN)
    out = _sc_embed(idx2, idxf, token_table, pos_table)
    return out.reshape(BATCH, SEQ, D)


# speedup vs baseline: 7.8846x; 1.2441x over previous
import functools

import jax
import jax.numpy as jnp
from jax import lax
from jax.experimental import pallas as pl
from jax.experimental.pallas import tpu as pltpu
from jax.experimental.pallas import tpu_sc as plsc

BATCH = 4096
SEQ = 200
VOCAB = 100000
D = 64
LANES = 16

NC = 2
NS = 16
NW = NC * NS

TOK = BATCH * SEQ
TOK_W = TOK // NW
CHUNK = 2 * SEQ
NCHUNK = TOK_W // CHUNK
GSUB = 8
GLEN = CHUNK // GSUB
IDX2_ROWS = TOK // GLEN
NBUF = 4


def _sc_body(idx2_hbm, idxf_hbm, table_hbm, pos_hbm, out_hbm, *s):
    idx2_b = s[0:NBUF]
    idxf_b = s[NBUF:2 * NBUF]
    pos_v = s[2 * NBUF]
    rows_b = s[2 * NBUF + 1:3 * NBUF + 1]
    gsem = s[3 * NBUF + 1:4 * NBUF + 1]
    osem = s[4 * NBUF + 1:5 * NBUF + 1]
    isem = s[5 * NBUF + 1:6 * NBUF + 1]
    mn_v = s[6 * NBUF + 1]

    wid = lax.axis_index("s") * NC + lax.axis_index("c")
    tok0 = wid * TOK_W

    def chunk_base(c):
        return pl.multiple_of(tok0 + c * CHUNK, CHUNK)

    def fire_idx(c, b):
        base = chunk_base(c)
        pltpu.async_copy(
            idx2_hbm.at[pl.ds(pl.multiple_of(base // GLEN, GSUB), GSUB)],
            idx2_b[b], isem[b])
        pltpu.async_copy(idxf_hbm.at[pl.ds(base, CHUNK)],
                         idxf_b[b].at[pl.ds(0, CHUNK)], isem[b])

    def wait_idx(b):
        pltpu.make_async_copy(idx2_hbm.at[pl.ds(0, GSUB)],
                              idx2_b[b], isem[b]).wait()
        pltpu.make_async_copy(idxf_hbm.at[pl.ds(0, CHUNK)],
                              idxf_b[b].at[pl.ds(0, CHUNK)], isem[b]).wait()

    def fire_gather(b):
        for j in range(GSUB):
            pltpu.async_copy(table_hbm.at[idx2_b[b].at[j]],
                             rows_b[b].at[pl.ds(j * GLEN, GLEN)], gsem[b])

    def wait_gather(b):
        pltpu.make_async_copy(out_hbm.at[pl.ds(0, CHUNK)],
                              rows_b[b], gsem[b]).wait()

    def fire_out(c, b):
        pltpu.async_copy(rows_b[b], out_hbm.at[pl.ds(chunk_base(c), CHUNK)],
                         osem[b])

    def wait_out(b):
        pltpu.make_async_copy(rows_b[b], out_hbm.at[pl.ds(0, CHUNK)],
                              osem[b]).wait()

    def compute(b):
        idxf_v = idxf_b[b]
        rows_v = rows_b[b]
        zero = jnp.zeros((LANES,), jnp.float32)

        mn = idxf_v[pl.ds(0, LANES)]
        for g in range(1, CHUNK // LANES):
            mn = jnp.minimum(mn, idxf_v[pl.ds(g * LANES, LANES)])
        mn_v[pl.ds(0, LANES)] = mn
        smin = mn_v[pl.ds(0, LANES)][0]
        for i in range(1, LANES):
            smin = jnp.minimum(smin, mn_v[pl.ds(i, LANES)][0])

        @pl.when(smin == 0)
        def _zero_pad_rows():
            def zrow(t, carry2):
                it = idxf_v[pl.ds(t, LANES)][0]

                @pl.when(it == 0)
                def _z():
                    for q in range(D // LANES):
                        rows_v[t, pl.ds(q * LANES, LANES)] = zero

                return carry2

            lax.fori_loop(0, CHUNK, zrow, 0)

        UNROLL = 8

        def pos_body(k, carry2):
            for u in range(UNROLL):
                t = k * UNROLL + u
                for q in range(D // LANES):
                    sl = pl.ds(q * LANES, LANES)
                    pv = pos_v[t, sl]
                    plsc.addupdate(rows_v.at[t, sl], pv)
                    plsc.addupdate(rows_v.at[t + SEQ, sl], pv)
            return carry2

        lax.fori_loop(0, SEQ // UNROLL, pos_body, 0)

    pltpu.sync_copy(pos_hbm, pos_v)

    fire_idx(0, 0)
    fire_idx(1, 1)
    wait_idx(0)
    fire_gather(0)

    def outer(k, carry):
        for b in range(NBUF):
            c = k * NBUF + b

            @pl.when(c + 2 < NCHUNK)
            def _s1():
                fire_idx(c + 2, (b + 2) % NBUF)

            @pl.when(c + 1 < NCHUNK)
            def _s2():
                @pl.when(c >= 3)
                def _drain():
                    wait_out((b + 1) % NBUF)

                wait_idx((b + 1) % NBUF)
                fire_gather((b + 1) % NBUF)

            wait_gather(b)
            compute(b)
            fire_out(c, b)
        return carry

    lax.fori_loop(0, NCHUNK // NBUF, outer, 0)

    for b in range(NBUF):
        wait_out(b)


@jax.jit
def _sc_embed(idx2, idxf, token_table, pos_table):
    mesh = plsc.VectorSubcoreMesh(core_axis_name="c", subcore_axis_name="s")
    run = functools.partial(
        pl.kernel,
        out_type=jax.ShapeDtypeStruct((TOK, D), jnp.float32),
        mesh=mesh,
        compiler_params=pltpu.CompilerParams(use_tc_tiling_on_sc=False),
        scratch_types=(
            [pltpu.VMEM((GSUB, GLEN), jnp.int32)] * NBUF
            + [pltpu.VMEM((CHUNK + LANES,), jnp.int32)] * NBUF
            + [pltpu.VMEM((SEQ, D), jnp.float32)]
            + [pltpu.VMEM((CHUNK, D), jnp.float32)] * NBUF
            + [pltpu.SemaphoreType.DMA] * (3 * NBUF)
            + [pltpu.VMEM((2 * LANES,), jnp.int32)]
        ),
    )(_sc_body)
    return run(idx2, idxf, token_table, pos_table)


def kernel(inputs, token_table, pos_table):
    idxf = inputs.reshape(TOK).astype(jnp.int32)
    idx2 = idxf.reshape(IDX2_ROWS, GLEN)
    out = _sc_embed(idx2, idxf, token_table, pos_table)
    return out.reshape(BATCH, SEQ, D)


# speedup vs baseline: 8.0731x; 1.0239x over previous
import functools

import jax
import jax.numpy as jnp
from jax import lax
from jax.experimental import pallas as pl
from jax.experimental.pallas import tpu as pltpu
from jax.experimental.pallas import tpu_sc as plsc

BATCH = 4096
SEQ = 200
VOCAB = 100000
D = 64
LANES = 16

NC = 2
NS = 16
NW = NC * NS

TOK = BATCH * SEQ
TOK_W = TOK // NW
CHUNK = 2 * SEQ
NCHUNK = TOK_W // CHUNK
GSUB = 5
GLEN = CHUNK // GSUB
NBUF = 4


def _sc_body(idxf_hbm, table_hbm, pos_hbm, out_hbm, *s):
    idxf_b = s[0:NBUF]
    pos_v = s[NBUF]
    rows_b = s[NBUF + 1:2 * NBUF + 1]
    gsem = s[2 * NBUF + 1:3 * NBUF + 1]
    osem = s[3 * NBUF + 1:4 * NBUF + 1]
    isem = s[4 * NBUF + 1:5 * NBUF + 1]
    mn_v = s[5 * NBUF + 1]

    wid = lax.axis_index("s") * NC + lax.axis_index("c")
    tok0 = wid * TOK_W

    def chunk_base(c):
        return pl.multiple_of(tok0 + c * CHUNK, CHUNK)

    def fire_idx(c, b):
        base = chunk_base(c)
        pltpu.async_copy(idxf_hbm.at[pl.ds(base, CHUNK)],
                         idxf_b[b].at[pl.ds(0, CHUNK)], isem[b])

    def wait_idx(b):
        pltpu.make_async_copy(idxf_hbm.at[pl.ds(0, CHUNK)],
                              idxf_b[b].at[pl.ds(0, CHUNK)], isem[b]).wait()

    def fire_gather(b):
        for j in range(GSUB):
            pltpu.async_copy(table_hbm.at[idxf_b[b].at[pl.ds(j * GLEN, GLEN)]],
                             rows_b[b].at[pl.ds(j * GLEN, GLEN)], gsem[b])

    def wait_gather(b):
        pltpu.make_async_copy(out_hbm.at[pl.ds(0, CHUNK)],
                              rows_b[b], gsem[b]).wait()

    def fire_out(c, b):
        pltpu.async_copy(rows_b[b], out_hbm.at[pl.ds(chunk_base(c), CHUNK)],
                         osem[b])

    def wait_out(b):
        pltpu.make_async_copy(rows_b[b], out_hbm.at[pl.ds(0, CHUNK)],
                              osem[b]).wait()

    def compute(b):
        idxf_v = idxf_b[b]
        rows_v = rows_b[b]
        zero = jnp.zeros((LANES,), jnp.float32)

        mn = idxf_v[pl.ds(0, LANES)]
        for g in range(1, CHUNK // LANES):
            mn = jnp.minimum(mn, idxf_v[pl.ds(g * LANES, LANES)])
        mn_v[pl.ds(0, LANES)] = mn
        smin = mn_v[pl.ds(0, LANES)][0]
        for i in range(1, LANES):
            smin = jnp.minimum(smin, mn_v[pl.ds(i, LANES)][0])

        @pl.when(smin == 0)
        def _zero_pad_rows():
            def zrow(t, carry2):
                it = idxf_v[pl.ds(t, LANES)][0]

                @pl.when(it == 0)
                def _z():
                    for q in range(D // LANES):
                        rows_v[t, pl.ds(q * LANES, LANES)] = zero

                return carry2

            lax.fori_loop(0, CHUNK, zrow, 0)

        UNROLL = 8

        def pos_body(k, carry2):
            for u in range(UNROLL):
                t = k * UNROLL + u
                for q in range(D // LANES):
                    sl = pl.ds(q * LANES, LANES)
                    pv = pos_v[t, sl]
                    plsc.addupdate(rows_v.at[t, sl], pv)
                    plsc.addupdate(rows_v.at[t + SEQ, sl], pv)
            return carry2

        lax.fori_loop(0, SEQ // UNROLL, pos_body, 0)

    pltpu.sync_copy(pos_hbm, pos_v)

    fire_idx(0, 0)
    fire_idx(1, 1)
    wait_idx(0)
    fire_gather(0)

    def outer(k, carry):
        for b in range(NBUF):
            c = k * NBUF + b

            @pl.when(c + 2 < NCHUNK)
            def _s1():
                fire_idx(c + 2, (b + 2) % NBUF)

            @pl.when(c + 1 < NCHUNK)
            def _s2():
                @pl.when(c >= 3)
                def _drain():
                    wait_out((b + 1) % NBUF)

                wait_idx((b + 1) % NBUF)
                fire_gather((b + 1) % NBUF)

            wait_gather(b)
            compute(b)
            fire_out(c, b)
        return carry

    lax.fori_loop(0, NCHUNK // NBUF, outer, 0)

    for b in range(NBUF):
        wait_out(b)


@jax.jit
def _sc_embed(idxf, token_table, pos_table):
    mesh = plsc.VectorSubcoreMesh(core_axis_name="c", subcore_axis_name="s")
    run = functools.partial(
        pl.kernel,
        out_type=jax.ShapeDtypeStruct((TOK, D), jnp.float32),
        mesh=mesh,
        compiler_params=pltpu.CompilerParams(use_tc_tiling_on_sc=False),
        scratch_types=(
            [pltpu.VMEM((CHUNK + LANES,), jnp.int32)] * NBUF
            + [pltpu.VMEM((SEQ, D), jnp.float32)]
            + [pltpu.VMEM((CHUNK, D), jnp.float32)] * NBUF
            + [pltpu.SemaphoreType.DMA] * (3 * NBUF)
            + [pltpu.VMEM((2 * LANES,), jnp.int32)]
        ),
    )(_sc_body)
    return run(idxf, token_table, pos_table)


def kernel(inputs, token_table, pos_table):
    idxf = inputs.reshape(TOK).astype(jnp.int32)
    out = _sc_embed(idxf, token_table, pos_table)
    return out.reshape(BATCH, SEQ, D)


# speedup vs baseline: 8.0827x; 1.0012x over previous
import functools

import jax
import jax.numpy as jnp
from jax import lax
from jax.experimental import pallas as pl
from jax.experimental.pallas import tpu as pltpu
from jax.experimental.pallas import tpu_sc as plsc

BATCH = 4096
SEQ = 200
VOCAB = 100000
D = 64
LANES = 16

NC = 2
NS = 16
NW = NC * NS

TOK = BATCH * SEQ
TOK_W = TOK // NW
CHUNK = 2 * SEQ
NCHUNK = TOK_W // CHUNK
GSPLIT = ((0, 104), (104, 96))
NBUF = 4


def _sc_body(idxf_hbm, table_hbm, pos_hbm, out_hbm, *s):
    idxf_b = s[0:NBUF]
    pos_v = s[NBUF]
    rows_b = s[NBUF + 1:2 * NBUF + 1]
    gsem = s[2 * NBUF + 1:3 * NBUF + 1]
    osem = s[3 * NBUF + 1:4 * NBUF + 1]
    isem = s[4 * NBUF + 1:5 * NBUF + 1]
    mn_v = s[5 * NBUF + 1]

    wid = lax.axis_index("s") * NC + lax.axis_index("c")
    tok0 = wid * TOK_W

    def chunk_base(c):
        return pl.multiple_of(tok0 + c * CHUNK, CHUNK)

    def fire_idx(c, b):
        base = chunk_base(c)
        pltpu.async_copy(idxf_hbm.at[pl.ds(base, CHUNK)],
                         idxf_b[b].at[pl.ds(0, CHUNK)], isem[b])

    def wait_idx(b):
        pltpu.make_async_copy(idxf_hbm.at[pl.ds(0, CHUNK)],
                              idxf_b[b].at[pl.ds(0, CHUNK)], isem[b]).wait()

    def fire_gather(b):
        for jb in range(2):
            for o, n in GSPLIT:
                pltpu.async_copy(
                    table_hbm.at[idxf_b[b].at[pl.ds(jb * SEQ + o, n)]],
                    rows_b[b].at[jb].at[pl.ds(o, n)], gsem[b])

    def wait_gather(b):
        pltpu.make_async_copy(out_hbm.at[pl.ds(0, 2)],
                              rows_b[b], gsem[b]).wait()

    def fire_out(c, b):
        row0 = pl.multiple_of(chunk_base(c) // SEQ, 2)
        pltpu.async_copy(rows_b[b], out_hbm.at[pl.ds(row0, 2)], osem[b])

    def wait_out(b):
        pltpu.make_async_copy(rows_b[b], out_hbm.at[pl.ds(0, 2)],
                              osem[b]).wait()

    def compute(b):
        idxf_v = idxf_b[b]
        rows_v = rows_b[b]
        zero = jnp.zeros((LANES,), jnp.float32)

        mn = idxf_v[pl.ds(0, LANES)]
        for g in range(1, CHUNK // LANES):
            mn = jnp.minimum(mn, idxf_v[pl.ds(g * LANES, LANES)])
        mn_v[pl.ds(0, LANES)] = mn
        smin = mn_v[pl.ds(0, LANES)][0]
        for i in range(1, LANES):
            smin = jnp.minimum(smin, mn_v[pl.ds(i, LANES)][0])

        @pl.when(smin == 0)
        def _zero_pad_rows():
            def zrow(t, carry2):
                for jb in range(2):
                    it = idxf_v[pl.ds(jb * SEQ + t, LANES)][0]

                    @pl.when(it == 0)
                    def _z():
                        for q in range(D // LANES):
                            rows_v[jb, t, pl.ds(q * LANES, LANES)] = zero

                return carry2

            lax.fori_loop(0, SEQ, zrow, 0)

        UNROLL = 8

        def pos_body(k, carry2):
            for u in range(UNROLL):
                t = k * UNROLL + u
                for q in range(D // LANES):
                    sl = pl.ds(q * LANES, LANES)
                    pv = pos_v[t, sl]
                    plsc.addupdate(rows_v.at[0, t, sl], pv)
                    plsc.addupdate(rows_v.at[1, t, sl], pv)
            return carry2

        lax.fori_loop(0, SEQ // UNROLL, pos_body, 0)

    pltpu.sync_copy(pos_hbm, pos_v)

    fire_idx(0, 0)
    fire_idx(1, 1)
    wait_idx(0)
    fire_gather(0)

    def outer(k, carry):
        for b in range(NBUF):
            c = k * NBUF + b

            @pl.when(c + 2 < NCHUNK)
            def _s1():
                fire_idx(c + 2, (b + 2) % NBUF)

            @pl.when(c + 1 < NCHUNK)
            def _s2():
                @pl.when(c >= 3)
                def _drain():
                    wait_out((b + 1) % NBUF)

                wait_idx((b + 1) % NBUF)
                fire_gather((b + 1) % NBUF)

            wait_gather(b)
            compute(b)
            fire_out(c, b)
        return carry

    lax.fori_loop(0, NCHUNK // NBUF, outer, 0)

    for b in range(NBUF):
        wait_out(b)


@jax.jit
def _sc_embed(idxf, token_table, pos_table):
    mesh = plsc.VectorSubcoreMesh(core_axis_name="c", subcore_axis_name="s")
    run = functools.partial(
        pl.kernel,
        out_type=jax.ShapeDtypeStruct((BATCH, SEQ, D), jnp.float32),
        mesh=mesh,
        compiler_params=pltpu.CompilerParams(use_tc_tiling_on_sc=False),
        scratch_types=(
            [pltpu.VMEM((CHUNK + LANES,), jnp.int32)] * NBUF
            + [pltpu.VMEM((SEQ, D), jnp.float32)]
            + [pltpu.VMEM((2, SEQ, D), jnp.float32)] * NBUF
            + [pltpu.SemaphoreType.DMA] * (3 * NBUF)
            + [pltpu.VMEM((2 * LANES,), jnp.int32)]
        ),
    )(_sc_body)
    return run(idxf, token_table, pos_table)


def kernel(inputs, token_table, pos_table):
    idxf = inputs.reshape(TOK).astype(jnp.int32)
    return _sc_embed(idxf, token_table, pos_table)


# speedup vs baseline: 8.4050x; 1.0399x over previous
import functools

import jax
import jax.numpy as jnp
from jax import lax
from jax.experimental import pallas as pl
from jax.experimental.pallas import tpu as pltpu
from jax.experimental.pallas import tpu_sc as plsc

BATCH = 4096
SEQ = 200
VOCAB = 100000
D = 64
LANES = 16

NC = 2
NS = 16
NW = NC * NS

BT = BATCH // NW
ST = SEQ // 8
NCHUNK = SEQ
NBUF = 4


def _sc_body(idx5_hbm, table_hbm, pos_hbm, outt_hbm, *s):
    rows_b = s[0:NBUF]
    idx_all = s[NBUF]
    pos_v = s[NBUF + 1]
    mn_v = s[NBUF + 2]
    gsem = s[NBUF + 3:2 * NBUF + 3]
    osem = s[2 * NBUF + 3:3 * NBUF + 3]

    wid = lax.axis_index("s") * NC + lax.axis_index("c")
    b0 = pl.multiple_of(wid * BT, BT)

    for st in range(ST):
        pltpu.sync_copy(idx5_hbm.at[st, wid], idx_all.at[st])
    pltpu.sync_copy(pos_hbm, pos_v)

    def fire_gather(sq, b):
        pltpu.async_copy(
            table_hbm.at[idx_all.at[sq // 8, lax.rem(sq, 8)]],
            rows_b[b], gsem[b])

    def wait_gather(b):
        pltpu.make_async_copy(table_hbm.at[pl.ds(0, BT)],
                              rows_b[b], gsem[b]).wait()

    def fire_out(sq, b):
        pltpu.async_copy(rows_b[b], outt_hbm.at[sq, pl.ds(b0, BT)], osem[b])

    def wait_out(b):
        pltpu.make_async_copy(rows_b[b], outt_hbm.at[0, pl.ds(0, BT)],
                              osem[b]).wait()

    def compute(sq, b):
        rows_v = rows_b[b]
        st = sq // 8
        sl = lax.rem(sq, 8)
        zero = jnp.zeros((LANES,), jnp.float32)

        mn = idx_all[st, sl, pl.ds(0, LANES)]
        for g in range(1, BT // LANES):
            mn = jnp.minimum(mn, idx_all[st, sl, pl.ds(g * LANES, LANES)])
        mn_v[pl.ds(0, LANES)] = mn
        smin = mn_v[pl.ds(0, LANES)][0]
        for i in range(1, LANES):
            smin = jnp.minimum(smin, mn_v[pl.ds(i, LANES)][0])

        @pl.when(smin == 0)
        def _zero_pad_rows():
            def zgroup(g, carry2):
                iv = idx_all[st, sl, pl.ds(g * LANES, LANES)]
                for i in range(LANES):
                    @pl.when(iv[i] == 0)
                    def _z():
                        for q in range(D // LANES):
                            rows_v[g * LANES + i,
                                   pl.ds(q * LANES, LANES)] = zero

                return carry2

            lax.fori_loop(0, BT // LANES, zgroup, 0)

        pos_q = [pos_v[sq, pl.ds(q * LANES, LANES)] for q in range(D // LANES)]

        UNROLL = 8

        def pos_body(k, carry2):
            for u in range(UNROLL):
                t = k * UNROLL + u
                for q in range(D // LANES):
                    plsc.addupdate(rows_v.at[t, pl.ds(q * LANES, LANES)],
                                   pos_q[q])
            return carry2

        lax.fori_loop(0, BT // UNROLL, pos_body, 0)

    fire_gather(0, 0)
    fire_gather(1, 1)

    def outer(k, carry):
        for b in range(NBUF):
            sq = k * NBUF + b

            @pl.when(sq + 2 < NCHUNK)
            def _prefetch():
                @pl.when(sq >= 2)
                def _drain():
                    wait_out((b + 2) % NBUF)

                fire_gather(sq + 2, (b + 2) % NBUF)

            wait_gather(b)
            compute(sq, b)
            fire_out(sq, b)
        return carry

    lax.fori_loop(0, NCHUNK // NBUF, outer, 0)

    for b in range(NBUF):
        wait_out(b)


@jax.jit
def _sc_embed(idx5, token_table, pos_table):
    mesh = plsc.VectorSubcoreMesh(core_axis_name="c", subcore_axis_name="s")
    run = functools.partial(
        pl.kernel,
        out_type=jax.ShapeDtypeStruct((SEQ, BATCH, D), jnp.float32),
        mesh=mesh,
        compiler_params=pltpu.CompilerParams(use_tc_tiling_on_sc=False),
        scratch_types=(
            [pltpu.VMEM((BT, D), jnp.float32)] * NBUF
            + [pltpu.VMEM((ST, 8, BT), jnp.int32)]
            + [pltpu.VMEM((SEQ, D), jnp.float32)]
            + [pltpu.VMEM((2 * LANES,), jnp.int32)]
            + [pltpu.SemaphoreType.DMA] * (2 * NBUF)
        ),
    )(_sc_body)
    return run(idx5, token_table, pos_table)


def kernel(inputs, token_table, pos_table):
    idx = inputs.astype(jnp.int32)
    idx5 = idx.T.reshape(ST, 8, NW, BT).transpose(0, 2, 1, 3)
    out_t = _sc_embed(idx5, token_table, pos_table)
    return out_t.transpose(1, 0, 2)
